# Initial kernel scaffold; baseline (speedup 1.0000x reference)
import jax
import jax.numpy as jnp
from jax.experimental import pallas as pl


def kernel(x, edge_index, edge_attr, W1a, b1a, W1b, b1b, W2a, b2a, W2b, b2b):
    raise NotImplementedError("write your pallas kernel here")


# speedup vs baseline: 3.1472x; 3.1472x over previous
import functools

import jax
import jax.numpy as jnp
from jax import lax
from jax.experimental import pallas as pl
from jax.experimental.pallas import tpu as pltpu
from jax.experimental.pallas import tpu_sc as plsc

_NC = 2
_NS = 16
_NW = _NC * _NS

_CH = 80
_NPAD = 10240


def _gather_rows(xw, row, E, D):
    per_w = E // _NW
    nch = per_w // _CH
    mesh = plsc.VectorSubcoreMesh(core_axis_name="c", subcore_axis_name="s")

    @functools.partial(
        pl.kernel,
        mesh=mesh,
        out_type=jax.ShapeDtypeStruct((E, D), jnp.float32),
        scratch_types=[
            pltpu.VMEM((_CH,), jnp.int32),
            pltpu.VMEM((_CH, D), jnp.float32),
            pltpu.SemaphoreType.DMA,
        ],
    )
    def k(xw_hbm, row_hbm, out_hbm, idx_v, rows_v, sem):
        wid = lax.axis_index("s") * _NC + lax.axis_index("c")
        base = wid * per_w

        def body(i, carry):
            off = base + i * _CH
            pltpu.sync_copy(row_hbm.at[pl.ds(off, _CH)], idx_v)
            pltpu.async_copy(xw_hbm.at[idx_v], rows_v, sem).wait()
            pltpu.sync_copy(rows_v, out_hbm.at[pl.ds(off, _CH)])
            return carry

        lax.fori_loop(0, nch, body, 0)

    return k(xw, row)


def _scatter_sums(h, col, E, D):
    per_w = E // _NW
    nch = per_w // _CH
    rpt = _NPAD // _NS
    mesh = plsc.VectorSubcoreMesh(core_axis_name="c", subcore_axis_name="s")

    @functools.partial(
        pl.kernel,
        mesh=mesh,
        out_type=jax.ShapeDtypeStruct((_NC, _NPAD, D), jnp.float32),
        scratch_types=[
            pltpu.VMEM((_CH,), jnp.int32),
            pltpu.VMEM((_CH, D), jnp.float32),
            pltpu.VMEM_SHARED((_NPAD, D), jnp.float32),
        ],
    )
    def k(h_hbm, col_hbm, sums_out, idx_v, rows_v, sums_sh):
        c = lax.axis_index("c")
        s = lax.axis_index("s")
        wid = s * _NC + c
        zeros16 = jnp.zeros((16,), jnp.float32)

        def fill_zero(i, carry):
            for j in range(D // 16):
                rows_v[i, pl.ds(j * 16, 16)] = zeros16
            return carry

        lax.fori_loop(0, _CH, fill_zero, 0)

        nbase = s * rpt
        for z in range(rpt // _CH):
            pltpu.sync_copy(rows_v, sums_sh.at[pl.ds(nbase + z * _CH, _CH)])
        plsc.subcore_barrier()

        base = wid * per_w

        def body(i, carry):
            off = base + i * _CH
            pltpu.sync_copy(col_hbm.at[pl.ds(off, _CH)], idx_v)
            pltpu.sync_copy(h_hbm.at[pl.ds(off, _CH)], rows_v)
            pltpu.sync_copy(rows_v, sums_sh.at[idx_v], add=True)
            return carry

        lax.fori_loop(0, nch, body, 0)
        plsc.subcore_barrier()

        pltpu.sync_copy(sums_sh.at[pl.ds(nbase, rpt)],
                        sums_out.at[c, pl.ds(nbase, rpt)])

    return k(h, col)


def _scatter_counts(col, E, D):
    per_w = E // _NW
    nch = per_w // _CH
    rpt = _NPAD // _NS
    mesh = plsc.VectorSubcoreMesh(core_axis_name="c", subcore_axis_name="s")

    @functools.partial(
        pl.kernel,
        mesh=mesh,
        out_type=jax.ShapeDtypeStruct((_NC, _NPAD, D), jnp.float32),
        scratch_types=[
            pltpu.VMEM((_CH,), jnp.int32),
            pltpu.VMEM((_CH, D), jnp.float32),
            pltpu.VMEM_SHARED((_NPAD, D), jnp.float32),
        ],
    )
    def k(col_hbm, cnt_out, idx_v, ones_v, cnt_sh):
        c = lax.axis_index("c")
        s = lax.axis_index("s")
        wid = s * _NC + c
        zeros16 = jnp.zeros((16,), jnp.float32)
        ones16 = jnp.ones((16,), jnp.float32)

        def fill_zero(i, carry):
            for j in range(D // 16):
                ones_v[i, pl.ds(j * 16, 16)] = zeros16
            return carry

        lax.fori_loop(0, _CH, fill_zero, 0)

        nbase = s * rpt
        for z in range(rpt // _CH):
            pltpu.sync_copy(ones_v, cnt_sh.at[pl.ds(nbase + z * _CH, _CH)])

        def fill_ones(i, carry):
            for j in range(D // 16):
                ones_v[i, pl.ds(j * 16, 16)] = ones16
            return carry

        lax.fori_loop(0, _CH, fill_ones, 0)
        plsc.subcore_barrier()

        base = wid * per_w

        def body(i, carry):
            off = base + i * _CH
            pltpu.sync_copy(col_hbm.at[pl.ds(off, _CH)], idx_v)
            pltpu.sync_copy(ones_v, cnt_sh.at[idx_v], add=True)
            return carry

        lax.fori_loop(0, nch, body, 0)
        plsc.subcore_barrier()

        pltpu.sync_copy(cnt_sh.at[pl.ds(nbase, rpt)],
                        cnt_out.at[c, pl.ds(nbase, rpt)])

    return k(col)


def _mm_body(x_ref, w_ref, o_ref):
    o_ref[...] = jnp.dot(x_ref[...], w_ref[...],
                         preferred_element_type=jnp.float32)


def _edge_mlp_body(xg_ref, ea_ref, w1e_ref, b1a_ref, w1b_ref, b1b_ref, o_ref):
    h1 = xg_ref[...] + jnp.dot(ea_ref[...], w1e_ref[...],
                               preferred_element_type=jnp.float32)
    h1 = jnp.maximum(h1 + b1a_ref[...], 0.0)
    h2 = jnp.dot(h1, w1b_ref[...], preferred_element_type=jnp.float32)
    o_ref[...] = jnp.maximum(h2 + b1b_ref[...], 0.0)


def _node_mlp_body(x_ref, p0_ref, p1_ref, c0_ref, c1_ref, w2x_ref, w2a_ref,
                   b2a_ref, w2b_ref, b2b_ref, o_ref):
    cnt = c0_ref[0, :, 0:1] + c1_ref[0, :, 0:1]
    agg = (p0_ref[0] + p1_ref[0]) / jnp.maximum(cnt, 1.0)
    hid = jnp.dot(x_ref[...], w2x_ref[...], preferred_element_type=jnp.float32)
    hid = hid + jnp.dot(agg, w2a_ref[...], preferred_element_type=jnp.float32)
    hid = jnp.maximum(hid + b2a_ref[...], 0.0)
    out = jnp.dot(hid, w2b_ref[...], preferred_element_type=jnp.float32)
    o_ref[...] = out + b2b_ref[...]


def kernel(x, edge_index, edge_attr, W1a, b1a, W1b, b1b, W2a, b2a, W2b, b2b):
    N, D = x.shape
    E = edge_index.shape[1]
    DE = edge_attr.shape[1]
    row = edge_index[0].astype(jnp.int32)
    col = edge_index[1].astype(jnp.int32)

    W1a_x = W1a[:D]
    W1a_e = W1a[D:]
    W2a_x = W2a[:D]
    W2a_a = W2a[D:]
    b1a2 = b1a.reshape(1, -1)
    b1b2 = b1b.reshape(1, -1)
    b2a2 = b2a.reshape(1, -1)
    b2b2 = b2b.reshape(1, -1)

    NB = 2000
    xw = pl.pallas_call(
        _mm_body,
        grid=(N // NB,),
        in_specs=[
            pl.BlockSpec((NB, D), lambda i: (i, 0)),
            pl.BlockSpec((D, D), lambda i: (0, 0)),
        ],
        out_specs=pl.BlockSpec((NB, D), lambda i: (i, 0)),
        out_shape=jax.ShapeDtypeStruct((N, D), jnp.float32),
    )(x, W1a_x)

    xg = _gather_rows(xw, row, E, D)

    EB = 2560
    h = pl.pallas_call(
        _edge_mlp_body,
        grid=(E // EB,),
        in_specs=[
            pl.BlockSpec((EB, D), lambda i: (i, 0)),
            pl.BlockSpec((EB, DE), lambda i: (i, 0)),
            pl.BlockSpec((DE, D), lambda i: (0, 0)),
            pl.BlockSpec((1, D), lambda i: (0, 0)),
            pl.BlockSpec((D, D), lambda i: (0, 0)),
            pl.BlockSpec((1, D), lambda i: (0, 0)),
        ],
        out_specs=pl.BlockSpec((EB, D), lambda i: (i, 0)),
        out_shape=jax.ShapeDtypeStruct((E, D), jnp.float32),
    )(xg, edge_attr, W1a_e, b1a2, W1b, b1b2)

    sums = _scatter_sums(h, col, E, D)
    cnt = _scatter_counts(col, E, D)

    out = pl.pallas_call(
        _node_mlp_body,
        grid=(N // NB,),
        in_specs=[
            pl.BlockSpec((NB, D), lambda i: (i, 0)),
            pl.BlockSpec((1, NB, D), lambda i: (0, i, 0)),
            pl.BlockSpec((1, NB, D), lambda i: (1, i, 0)),
            pl.BlockSpec((1, NB, D), lambda i: (0, i, 0)),
            pl.BlockSpec((1, NB, D), lambda i: (1, i, 0)),
            pl.BlockSpec((D, D), lambda i: (0, 0)),
            pl.BlockSpec((D, D), lambda i: (0, 0)),
            pl.BlockSpec((1, D), lambda i: (0, 0)),
            pl.BlockSpec((D, D), lambda i: (0, 0)),
            pl.BlockSpec((1, D), lambda i: (0, 0)),
        ],
        out_specs=pl.BlockSpec((NB, D), lambda i: (i, 0)),
        out_shape=jax.ShapeDtypeStruct((N, D), jnp.float32),
    )(x, sums, sums, cnt, cnt, W2a_x, W2a_a, b2a2, W2b, b2b2)
    return out


# speedup vs baseline: 3.4635x; 1.1005x over previous
import functools

import jax
import jax.numpy as jnp
from jax import lax
from jax.experimental import pallas as pl
from jax.experimental.pallas import tpu as pltpu
from jax.experimental.pallas import tpu_sc as plsc

_NC = 2
_NS = 16
_NW = _NC * _NS

_GCH = 80
_SCH = 40
_CCH = 80
_NPAD = 10240


def _gather_rows(xw_pad, row, E, D):
    per_w = E // _NW
    nch = per_w // _GCH
    rpt = _NPAD // _NS
    mesh = plsc.VectorSubcoreMesh(core_axis_name="c", subcore_axis_name="s")

    @functools.partial(
        pl.kernel,
        mesh=mesh,
        out_type=jax.ShapeDtypeStruct((E, D), jnp.float32),
        scratch_types=[
            pltpu.VMEM((_GCH,), jnp.int32),
            pltpu.VMEM((_GCH,), jnp.int32),
            pltpu.VMEM((_GCH, D), jnp.float32),
            pltpu.VMEM((_GCH, D), jnp.float32),
            pltpu.VMEM_SHARED((_NPAD, D), jnp.float32),
            pltpu.SemaphoreType.DMA,
            pltpu.SemaphoreType.DMA,
            pltpu.SemaphoreType.DMA,
        ],
    )
    def k(xw_hbm, row_hbm, out_hbm, idx0, idx1, r0, r1, xw_sh,
          semg, semw0, semw1):
        c = lax.axis_index("c")
        s = lax.axis_index("s")
        wid = s * _NC + c

        pltpu.sync_copy(xw_hbm.at[pl.ds(s * rpt, rpt)],
                        xw_sh.at[pl.ds(s * rpt, rpt)])
        plsc.subcore_barrier()

        base = wid * per_w
        idxs = (idx0, idx1)
        rows = (r0, r1)
        semws = (semw0, semw1)

        def do_chunk(kk, b):
            off = base + kk * _GCH
            pltpu.sync_copy(row_hbm.at[pl.ds(off, _GCH)], idxs[b])
            pltpu.async_copy(xw_sh.at[idxs[b]], rows[b], semg).wait()
            pltpu.async_copy(rows[b], out_hbm.at[pl.ds(off, _GCH)], semws[b])

        def wb_wait(b):
            pltpu.make_async_copy(rows[b], out_hbm.at[pl.ds(base, _GCH)],
                                  semws[b]).wait()

        do_chunk(0, 0)
        do_chunk(1, 1)

        def body(j, carry):
            wb_wait(0)
            do_chunk(2 * j, 0)
            wb_wait(1)
            do_chunk(2 * j + 1, 1)
            return carry

        lax.fori_loop(1, nch // 2, body, 0)
        wb_wait(0)
        do_chunk(nch - 1, 0)
        wb_wait(1)
        wb_wait(0)

    return k(xw_pad, row)


def _scatter_sums(h, col, E, D):
    per_w = E // _NW
    nch = per_w // _SCH
    rpt = _NPAD // _NS
    mesh = plsc.VectorSubcoreMesh(core_axis_name="c", subcore_axis_name="s")

    @functools.partial(
        pl.kernel,
        mesh=mesh,
        out_type=jax.ShapeDtypeStruct((_NC, _NPAD, D), jnp.float32),
        scratch_types=[
            pltpu.VMEM((_SCH,), jnp.int32),
            pltpu.VMEM((_SCH,), jnp.int32),
            pltpu.VMEM((_SCH, D), jnp.float32),
            pltpu.VMEM((_SCH, D), jnp.float32),
            pltpu.VMEM_SHARED((_NPAD, D), jnp.float32),
            pltpu.SemaphoreType.DMA,
            pltpu.SemaphoreType.DMA,
            pltpu.SemaphoreType.DMA,
        ],
    )
    def k(h_hbm, col_hbm, sums_out, idx0, idx1, r0, r1, sums_sh,
          semh, sema0, sema1):
        c = lax.axis_index("c")
        s = lax.axis_index("s")
        wid = s * _NC + c
        zeros16 = jnp.zeros((16,), jnp.float32)
        idxs = (idx0, idx1)
        rows = (r0, r1)
        semas = (sema0, sema1)

        def fill_zero(i, carry):
            for j in range(D // 16):
                r0[i, pl.ds(j * 16, 16)] = zeros16
            return carry

        lax.fori_loop(0, _SCH, fill_zero, 0)

        nbase = s * rpt
        for z in range(rpt // _SCH):
            pltpu.sync_copy(r0, sums_sh.at[pl.ds(nbase + z * _SCH, _SCH)])
        plsc.subcore_barrier()

        base = wid * per_w

        def do_chunk(kk, b):
            off = base + kk * _SCH
            pltpu.sync_copy(col_hbm.at[pl.ds(off, _SCH)], idxs[b])
            pltpu.async_copy(h_hbm.at[pl.ds(off, _SCH)], rows[b], semh).wait()
            pltpu.async_copy(rows[b], sums_sh.at[idxs[b]], semas[b], add=True)

        def add_wait(b):
            pltpu.make_async_copy(rows[b], sums_sh.at[idxs[b]],
                                  semas[b]).wait()

        do_chunk(0, 0)
        do_chunk(1, 1)

        def body(j, carry):
            add_wait(0)
            do_chunk(2 * j, 0)
            add_wait(1)
            do_chunk(2 * j + 1, 1)
            return carry

        lax.fori_loop(1, nch // 2, body, 0)
        add_wait(0)
        add_wait(1)
        plsc.subcore_barrier()
        pltpu.sync_copy(sums_sh.at[pl.ds(nbase, rpt)],
                        sums_out.at[c, pl.ds(nbase, rpt)])

    return k(h, col)


def _scatter_counts(col, E, D):
    per_w = E // _NW
    nch = per_w // _CCH
    rpt = _NPAD // _NS
    mesh = plsc.VectorSubcoreMesh(core_axis_name="c", subcore_axis_name="s")

    @functools.partial(
        pl.kernel,
        mesh=mesh,
        out_type=jax.ShapeDtypeStruct((_NC, _NPAD, D), jnp.float32),
        scratch_types=[
            pltpu.VMEM((_CCH,), jnp.int32),
            pltpu.VMEM((_CCH,), jnp.int32),
            pltpu.VMEM((_CCH, D), jnp.float32),
            pltpu.VMEM_SHARED((_NPAD, D), jnp.float32),
            pltpu.SemaphoreType.DMA,
            pltpu.SemaphoreType.DMA,
        ],
    )
    def k(col_hbm, cnt_out, idx0, idx1, ones_v, cnt_sh, sema0, sema1):
        c = lax.axis_index("c")
        s = lax.axis_index("s")
        wid = s * _NC + c
        zeros16 = jnp.zeros((16,), jnp.float32)
        ones16 = jnp.ones((16,), jnp.float32)
        idxs = (idx0, idx1)
        semas = (sema0, sema1)

        def fill_zero(i, carry):
            for j in range(D // 16):
                ones_v[i, pl.ds(j * 16, 16)] = zeros16
            return carry

        lax.fori_loop(0, _CCH, fill_zero, 0)

        nbase = s * rpt
        for z in range(rpt // _CCH):
            pltpu.sync_copy(ones_v, cnt_sh.at[pl.ds(nbase + z * _CCH, _CCH)])

        def fill_ones(i, carry):
            for j in range(D // 16):
                ones_v[i, pl.ds(j * 16, 16)] = ones16
            return carry

        lax.fori_loop(0, _CCH, fill_ones, 0)
        plsc.subcore_barrier()

        base = wid * per_w

        def do_chunk(kk, b):
            off = base + kk * _CCH
            pltpu.sync_copy(col_hbm.at[pl.ds(off, _CCH)], idxs[b])
            pltpu.async_copy(ones_v, cnt_sh.at[idxs[b]], semas[b], add=True)

        def add_wait(b):
            pltpu.make_async_copy(ones_v, cnt_sh.at[idxs[b]],
                                  semas[b]).wait()

        do_chunk(0, 0)
        do_chunk(1, 1)

        def body(j, carry):
            add_wait(0)
            do_chunk(2 * j, 0)
            add_wait(1)
            do_chunk(2 * j + 1, 1)
            return carry

        lax.fori_loop(1, nch // 2, body, 0)
        add_wait(0)
        do_chunk(nch - 1, 0)
        add_wait(1)
        add_wait(0)
        plsc.subcore_barrier()
        pltpu.sync_copy(cnt_sh.at[pl.ds(nbase, rpt)],
                        cnt_out.at[c, pl.ds(nbase, rpt)])

    return k(col)


def _mm_body(x_ref, w_ref, o_ref):
    o_ref[...] = jnp.dot(x_ref[...], w_ref[...],
                         preferred_element_type=jnp.float32)


def _edge_mlp_body(xg_ref, ea_ref, w1e_ref, b1a_ref, w1b_ref, b1b_ref, o_ref):
    h1 = xg_ref[...] + jnp.dot(ea_ref[...], w1e_ref[...],
                               preferred_element_type=jnp.float32)
    h1 = jnp.maximum(h1 + b1a_ref[...], 0.0)
    h2 = jnp.dot(h1, w1b_ref[...], preferred_element_type=jnp.float32)
    o_ref[...] = jnp.maximum(h2 + b1b_ref[...], 0.0)


def _node_mlp_body(x_ref, p0_ref, p1_ref, c0_ref, c1_ref, w2x_ref, w2a_ref,
                   b2a_ref, w2b_ref, b2b_ref, o_ref):
    cnt = c0_ref[0, :, 0:1] + c1_ref[0, :, 0:1]
    agg = (p0_ref[0] + p1_ref[0]) / jnp.maximum(cnt, 1.0)
    hid = jnp.dot(x_ref[...], w2x_ref[...], preferred_element_type=jnp.float32)
    hid = hid + jnp.dot(agg, w2a_ref[...], preferred_element_type=jnp.float32)
    hid = jnp.maximum(hid + b2a_ref[...], 0.0)
    out = jnp.dot(hid, w2b_ref[...], preferred_element_type=jnp.float32)
    o_ref[...] = out + b2b_ref[...]


def kernel(x, edge_index, edge_attr, W1a, b1a, W1b, b1b, W2a, b2a, W2b, b2b):
    N, D = x.shape
    E = edge_index.shape[1]
    DE = edge_attr.shape[1]
    row = edge_index[0].astype(jnp.int32)
    col = edge_index[1].astype(jnp.int32)

    W1a_x = W1a[:D]
    W1a_e = W1a[D:]
    W2a_x = W2a[:D]
    W2a_a = W2a[D:]
    b1a2 = b1a.reshape(1, -1)
    b1b2 = b1b.reshape(1, -1)
    b2a2 = b2a.reshape(1, -1)
    b2b2 = b2b.reshape(1, -1)

    cnt = _scatter_counts(col, E, D)

    x_pad = jnp.concatenate(
        [x, jnp.zeros((_NPAD - N, D), jnp.float32)], axis=0)
    PB = _NPAD // 8
    xw = pl.pallas_call(
        _mm_body,
        grid=(8,),
        in_specs=[
            pl.BlockSpec((PB, D), lambda i: (i, 0)),
            pl.BlockSpec((D, D), lambda i: (0, 0)),
        ],
        out_specs=pl.BlockSpec((PB, D), lambda i: (i, 0)),
        out_shape=jax.ShapeDtypeStruct((_NPAD, D), jnp.float32),
    )(x_pad, W1a_x)

    xg = _gather_rows(xw, row, E, D)

    EB = 2560
    h = pl.pallas_call(
        _edge_mlp_body,
        grid=(E // EB,),
        in_specs=[
            pl.BlockSpec((EB, D), lambda i: (i, 0)),
            pl.BlockSpec((EB, DE), lambda i: (i, 0)),
            pl.BlockSpec((DE, D), lambda i: (0, 0)),
            pl.BlockSpec((1, D), lambda i: (0, 0)),
            pl.BlockSpec((D, D), lambda i: (0, 0)),
            pl.BlockSpec((1, D), lambda i: (0, 0)),
        ],
        out_specs=pl.BlockSpec((EB, D), lambda i: (i, 0)),
        out_shape=jax.ShapeDtypeStruct((E, D), jnp.float32),
    )(xg, edge_attr, W1a_e, b1a2, W1b, b1b2)

    sums = _scatter_sums(h, col, E, D)

    NB = 2000
    out = pl.pallas_call(
        _node_mlp_body,
        grid=(N // NB,),
        in_specs=[
            pl.BlockSpec((NB, D), lambda i: (i, 0)),
            pl.BlockSpec((1, NB, D), lambda i: (0, i, 0)),
            pl.BlockSpec((1, NB, D), lambda i: (1, i, 0)),
            pl.BlockSpec((1, NB, D), lambda i: (0, i, 0)),
            pl.BlockSpec((1, NB, D), lambda i: (1, i, 0)),
            pl.BlockSpec((D, D), lambda i: (0, 0)),
            pl.BlockSpec((D, D), lambda i: (0, 0)),
            pl.BlockSpec((1, D), lambda i: (0, 0)),
            pl.BlockSpec((D, D), lambda i: (0, 0)),
            pl.BlockSpec((1, D), lambda i: (0, 0)),
        ],
        out_specs=pl.BlockSpec((NB, D), lambda i: (i, 0)),
        out_shape=jax.ShapeDtypeStruct((N, D), jnp.float32),
    )(x, sums, sums, cnt, cnt, W2a_x, W2a_a, b2a2, W2b, b2b2)
    return out


# speedup vs baseline: 4.0145x; 1.1591x over previous
import functools

import jax
import jax.numpy as jnp
from jax import lax
from jax.experimental import pallas as pl
from jax.experimental.pallas import tpu as pltpu
from jax.experimental.pallas import tpu_sc as plsc

_NC = 2
_NS = 16
_NW = _NC * _NS

_GCH = 80
_SCH = 40
_CCH = 80
_NPAD = 10240


def _gather_rows(xw_pad, row, E, D):
    per_w = E // _NW
    nch = per_w // _GCH
    rpt = _NPAD // _NS
    mesh = plsc.VectorSubcoreMesh(core_axis_name="c", subcore_axis_name="s")

    @functools.partial(
        pl.kernel,
        mesh=mesh,
        out_type=jax.ShapeDtypeStruct((E, D), jnp.float32),
        scratch_types=[
            pltpu.VMEM((_GCH,), jnp.int32),
            pltpu.VMEM((_GCH,), jnp.int32),
            pltpu.VMEM((_GCH, D), jnp.float32),
            pltpu.VMEM((_GCH, D), jnp.float32),
            pltpu.VMEM_SHARED((_NPAD, D), jnp.float32),
            pltpu.SemaphoreType.DMA,
            pltpu.SemaphoreType.DMA,
            pltpu.SemaphoreType.DMA,
        ],
    )
    def k(xw_hbm, row_hbm, out_hbm, idx0, idx1, r0, r1, xw_sh,
          semg, semw0, semw1):
        c = lax.axis_index("c")
        s = lax.axis_index("s")
        wid = s * _NC + c

        pltpu.sync_copy(xw_hbm.at[pl.ds(s * rpt, rpt)],
                        xw_sh.at[pl.ds(s * rpt, rpt)])
        plsc.subcore_barrier()

        base = wid * per_w
        idxs = (idx0, idx1)
        rows = (r0, r1)
        semws = (semw0, semw1)

        def do_chunk(kk, b):
            off = base + kk * _GCH
            pltpu.sync_copy(row_hbm.at[pl.ds(off, _GCH)], idxs[b])
            pltpu.async_copy(xw_sh.at[idxs[b]], rows[b], semg).wait()
            pltpu.async_copy(rows[b], out_hbm.at[pl.ds(off, _GCH)], semws[b])

        def wb_wait(b):
            pltpu.make_async_copy(rows[b], out_hbm.at[pl.ds(base, _GCH)],
                                  semws[b]).wait()

        do_chunk(0, 0)
        do_chunk(1, 1)

        def body(j, carry):
            wb_wait(0)
            do_chunk(2 * j, 0)
            wb_wait(1)
            do_chunk(2 * j + 1, 1)
            return carry

        lax.fori_loop(1, nch // 2, body, 0)
        wb_wait(0)
        do_chunk(nch - 1, 0)
        wb_wait(1)
        wb_wait(0)

    return k(xw_pad, row)


def _scatter_sums(h, col2, E, D):
    per_w = E // _NW
    nch = per_w // _SCH
    rpt = _NPAD // _NS
    mesh = plsc.VectorSubcoreMesh(core_axis_name="c", subcore_axis_name="s")

    @functools.partial(
        pl.kernel,
        mesh=mesh,
        out_type=jax.ShapeDtypeStruct((_NC, _NPAD, D), jnp.float32),
        scratch_types=[
            pltpu.VMEM((nch, _SCH), jnp.int32),
            pltpu.VMEM((_SCH, D), jnp.float32),
            pltpu.VMEM((_SCH, D), jnp.float32),
            pltpu.VMEM_SHARED((_NPAD, D), jnp.float32),
            pltpu.SemaphoreType.DMA,
            pltpu.SemaphoreType.DMA,
            pltpu.SemaphoreType.DMA,
        ],
    )
    def k(h_hbm, col_hbm, sums_out, idx_all, r0, r1, sums_sh,
          semh, sema0, sema1):
        c = lax.axis_index("c")
        s = lax.axis_index("s")
        wid = s * _NC + c
        zeros16 = jnp.zeros((16,), jnp.float32)
        rows = (r0, r1)
        semas = (sema0, sema1)

        pltpu.sync_copy(col_hbm.at[wid], idx_all)

        def fill_zero(i, carry):
            for j in range(D // 16):
                r0[i, pl.ds(j * 16, 16)] = zeros16
            return carry

        lax.fori_loop(0, _SCH, fill_zero, 0)

        nbase = s * rpt
        for z in range(rpt // _SCH):
            pltpu.sync_copy(r0, sums_sh.at[pl.ds(nbase + z * _SCH, _SCH)])
        plsc.subcore_barrier()

        base = wid * per_w

        def do_chunk(kk, b):
            off = base + kk * _SCH
            pltpu.async_copy(h_hbm.at[pl.ds(off, _SCH)], rows[b], semh).wait()
            pltpu.async_copy(rows[b], sums_sh.at[idx_all.at[kk]],
                             semas[b], add=True)

        def add_wait(b):
            pltpu.make_async_copy(rows[b], sums_sh.at[idx_all.at[0]],
                                  semas[b]).wait()

        do_chunk(0, 0)
        do_chunk(1, 1)

        def body(j, carry):
            add_wait(0)
            do_chunk(2 * j, 0)
            add_wait(1)
            do_chunk(2 * j + 1, 1)
            return carry

        lax.fori_loop(1, nch // 2, body, 0)
        add_wait(0)
        add_wait(1)
        plsc.subcore_barrier()
        pltpu.sync_copy(sums_sh.at[pl.ds(nbase, rpt)],
                        sums_out.at[c, pl.ds(nbase, rpt)])

    return k(h, col2)


def _scatter_counts(col, E, D):
    per_w = E // _NW
    nch = per_w // _CCH
    rpt = _NPAD // _NS
    mesh = plsc.VectorSubcoreMesh(core_axis_name="c", subcore_axis_name="s")

    @functools.partial(
        pl.kernel,
        mesh=mesh,
        out_type=jax.ShapeDtypeStruct((_NC, _NPAD, D), jnp.float32),
        scratch_types=[
            pltpu.VMEM((nch, _CCH), jnp.int32),
            pltpu.VMEM((_CCH, D), jnp.float32),
            pltpu.VMEM_SHARED((_NPAD, D), jnp.float32),
            pltpu.SemaphoreType.DMA,
            pltpu.SemaphoreType.DMA,
        ],
    )
    def k(col_hbm, cnt_out, idx_all, ones_v, cnt_sh, sema0, sema1):
        c = lax.axis_index("c")
        s = lax.axis_index("s")
        wid = s * _NC + c
        zeros16 = jnp.zeros((16,), jnp.float32)
        ones16 = jnp.ones((16,), jnp.float32)
        semas = (sema0, sema1)

        pltpu.sync_copy(col_hbm.at[wid], idx_all)

        def fill_zero(i, carry):
            for j in range(D // 16):
                ones_v[i, pl.ds(j * 16, 16)] = zeros16
            return carry

        lax.fori_loop(0, _CCH, fill_zero, 0)

        nbase = s * rpt
        for z in range(rpt // _CCH):
            pltpu.sync_copy(ones_v, cnt_sh.at[pl.ds(nbase + z * _CCH, _CCH)])

        def fill_ones(i, carry):
            for j in range(D // 16):
                ones_v[i, pl.ds(j * 16, 16)] = ones16
            return carry

        lax.fori_loop(0, _CCH, fill_ones, 0)
        plsc.subcore_barrier()

        def do_chunk(kk, b):
            pltpu.async_copy(ones_v, cnt_sh.at[idx_all.at[kk]],
                             semas[b], add=True)

        def add_wait(b):
            pltpu.make_async_copy(ones_v, cnt_sh.at[idx_all.at[0]],
                                  semas[b]).wait()

        do_chunk(0, 0)
        do_chunk(1, 1)

        def body(j, carry):
            add_wait(0)
            do_chunk(2 * j, 0)
            add_wait(1)
            do_chunk(2 * j + 1, 1)
            return carry

        lax.fori_loop(1, nch // 2, body, 0)
        add_wait(0)
        do_chunk(nch - 1, 0)
        add_wait(1)
        add_wait(0)
        plsc.subcore_barrier()
        pltpu.sync_copy(cnt_sh.at[pl.ds(nbase, rpt)],
                        cnt_out.at[c, pl.ds(nbase, rpt)])

    return k(col)


def _mm_body(x_ref, w_ref, o_ref):
    o_ref[...] = jnp.dot(x_ref[...], w_ref[...],
                         preferred_element_type=jnp.float32)


def _edge_mlp_body(xg_ref, ea_ref, w1e_ref, b1a_ref, w1b_ref, b1b_ref, o_ref):
    h1 = xg_ref[...] + jnp.dot(ea_ref[...], w1e_ref[...],
                               preferred_element_type=jnp.float32)
    h1 = jnp.maximum(h1 + b1a_ref[...], 0.0)
    h2 = jnp.dot(h1, w1b_ref[...], preferred_element_type=jnp.float32)
    o_ref[...] = jnp.maximum(h2 + b1b_ref[...], 0.0)


def _node_mlp_body(x_ref, p0_ref, p1_ref, c0_ref, c1_ref, w2x_ref, w2a_ref,
                   b2a_ref, w2b_ref, b2b_ref, o_ref):
    cnt = c0_ref[0, :, 0:1] + c1_ref[0, :, 0:1]
    agg = (p0_ref[0] + p1_ref[0]) / jnp.maximum(cnt, 1.0)
    hid = jnp.dot(x_ref[...], w2x_ref[...], preferred_element_type=jnp.float32)
    hid = hid + jnp.dot(agg, w2a_ref[...], preferred_element_type=jnp.float32)
    hid = jnp.maximum(hid + b2a_ref[...], 0.0)
    out = jnp.dot(hid, w2b_ref[...], preferred_element_type=jnp.float32)
    o_ref[...] = out + b2b_ref[...]


def kernel(x, edge_index, edge_attr, W1a, b1a, W1b, b1b, W2a, b2a, W2b, b2b):
    N, D = x.shape
    E = edge_index.shape[1]
    DE = edge_attr.shape[1]
    row = edge_index[0].astype(jnp.int32)
    col = edge_index[1].astype(jnp.int32)

    W1a_x = W1a[:D]
    W1a_e = W1a[D:]
    W2a_x = W2a[:D]
    W2a_a = W2a[D:]
    b1a2 = b1a.reshape(1, -1)
    b1b2 = b1b.reshape(1, -1)
    b2a2 = b2a.reshape(1, -1)
    b2b2 = b2b.reshape(1, -1)

    col_c = col.reshape(_NW, E // _NW // _CCH, _CCH)
    col_s = col.reshape(_NW, E // _NW // _SCH, _SCH)
    cnt = _scatter_counts(col_c, E, D)

    x_pad = jnp.concatenate(
        [x, jnp.zeros((_NPAD - N, D), jnp.float32)], axis=0)
    PB = _NPAD // 8
    xw = pl.pallas_call(
        _mm_body,
        grid=(8,),
        in_specs=[
            pl.BlockSpec((PB, D), lambda i: (i, 0)),
            pl.BlockSpec((D, D), lambda i: (0, 0)),
        ],
        out_specs=pl.BlockSpec((PB, D), lambda i: (i, 0)),
        out_shape=jax.ShapeDtypeStruct((_NPAD, D), jnp.float32),
    )(x_pad, W1a_x)

    xg = _gather_rows(xw, row, E, D)

    EB = 2560
    h = pl.pallas_call(
        _edge_mlp_body,
        grid=(E // EB,),
        in_specs=[
            pl.BlockSpec((EB, D), lambda i: (i, 0)),
            pl.BlockSpec((EB, DE), lambda i: (i, 0)),
            pl.BlockSpec((DE, D), lambda i: (0, 0)),
            pl.BlockSpec((1, D), lambda i: (0, 0)),
            pl.BlockSpec((D, D), lambda i: (0, 0)),
            pl.BlockSpec((1, D), lambda i: (0, 0)),
        ],
        out_specs=pl.BlockSpec((EB, D), lambda i: (i, 0)),
        out_shape=jax.ShapeDtypeStruct((E, D), jnp.float32),
    )(xg, edge_attr, W1a_e, b1a2, W1b, b1b2)

    sums = _scatter_sums(h, col_s, E, D)

    NB = 2000
    out = pl.pallas_call(
        _node_mlp_body,
        grid=(N // NB,),
        in_specs=[
            pl.BlockSpec((NB, D), lambda i: (i, 0)),
            pl.BlockSpec((1, NB, D), lambda i: (0, i, 0)),
            pl.BlockSpec((1, NB, D), lambda i: (1, i, 0)),
            pl.BlockSpec((1, NB, D), lambda i: (0, i, 0)),
            pl.BlockSpec((1, NB, D), lambda i: (1, i, 0)),
            pl.BlockSpec((D, D), lambda i: (0, 0)),
            pl.BlockSpec((D, D), lambda i: (0, 0)),
            pl.BlockSpec((1, D), lambda i: (0, 0)),
            pl.BlockSpec((D, D), lambda i: (0, 0)),
            pl.BlockSpec((1, D), lambda i: (0, 0)),
        ],
        out_specs=pl.BlockSpec((NB, D), lambda i: (i, 0)),
        out_shape=jax.ShapeDtypeStruct((N, D), jnp.float32),
    )(x, sums, sums, cnt, cnt, W2a_x, W2a_a, b2a2, W2b, b2b2)
    return out


# speedup vs baseline: 4.1059x; 1.0228x over previous
import functools

import jax
import jax.numpy as jnp
from jax import lax
from jax.experimental import pallas as pl
from jax.experimental.pallas import tpu as pltpu
from jax.experimental.pallas import tpu_sc as plsc

_NC = 2
_NS = 16
_NW = _NC * _NS

_GCH = 40
_SCH = 40
_CCH = 80
_NPAD = 10240


def _gather_rows(xw_pad, row2, E, D):
    per_w = E // _NW
    nch = per_w // _GCH
    rpt = _NPAD // _NS
    mesh = plsc.VectorSubcoreMesh(core_axis_name="c", subcore_axis_name="s")

    @functools.partial(
        pl.kernel,
        mesh=mesh,
        out_type=jax.ShapeDtypeStruct((E, D), jnp.float32),
        scratch_types=[
            pltpu.VMEM((nch, _GCH), jnp.int32),
            pltpu.VMEM((_GCH, D), jnp.float32),
            pltpu.VMEM((_GCH, D), jnp.float32),
            pltpu.VMEM_SHARED((_NPAD, D), jnp.float32),
            pltpu.SemaphoreType.DMA,
            pltpu.SemaphoreType.DMA,
            pltpu.SemaphoreType.DMA,
        ],
    )
    def k(xw_hbm, row_hbm, out_hbm, idx_all, r0, r1, xw_sh,
          semg, semw0, semw1):
        c = lax.axis_index("c")
        s = lax.axis_index("s")
        wid = s * _NC + c

        pltpu.sync_copy(row_hbm.at[wid], idx_all)
        pltpu.sync_copy(xw_hbm.at[pl.ds(s * rpt, rpt)],
                        xw_sh.at[pl.ds(s * rpt, rpt)])
        plsc.subcore_barrier()

        base = wid * per_w
        rows = (r0, r1)
        semws = (semw0, semw1)

        def do_chunk(kk, b):
            off = base + kk * _GCH
            pltpu.async_copy(xw_sh.at[idx_all.at[kk]], rows[b], semg).wait()
            pltpu.async_copy(rows[b], out_hbm.at[pl.ds(off, _GCH)], semws[b])

        def wb_wait(b):
            pltpu.make_async_copy(rows[b], out_hbm.at[pl.ds(base, _GCH)],
                                  semws[b]).wait()

        do_chunk(0, 0)
        do_chunk(1, 1)

        def body(j, carry):
            wb_wait(0)
            do_chunk(2 * j, 0)
            wb_wait(1)
            do_chunk(2 * j + 1, 1)
            return carry

        lax.fori_loop(1, nch // 2, body, 0)
        wb_wait(0)
        wb_wait(1)

    return k(xw_pad, row2)


def _scatter_sums(h, col2, E, D):
    per_w = E // _NW
    nch = per_w // _SCH
    rpt = _NPAD // _NS
    mesh = plsc.VectorSubcoreMesh(core_axis_name="c", subcore_axis_name="s")

    @functools.partial(
        pl.kernel,
        mesh=mesh,
        out_type=jax.ShapeDtypeStruct((_NC, _NPAD, D), jnp.float32),
        scratch_types=[
            pltpu.VMEM((nch, _SCH), jnp.int32),
            pltpu.VMEM((_SCH, D), jnp.float32),
            pltpu.VMEM((_SCH, D), jnp.float32),
            pltpu.VMEM_SHARED((_NPAD, D), jnp.float32),
            pltpu.SemaphoreType.DMA,
            pltpu.SemaphoreType.DMA,
            pltpu.SemaphoreType.DMA,
        ],
    )
    def k(h_hbm, col_hbm, sums_out, idx_all, r0, r1, sums_sh,
          semh, sema0, sema1):
        c = lax.axis_index("c")
        s = lax.axis_index("s")
        wid = s * _NC + c
        zeros16 = jnp.zeros((16,), jnp.float32)
        rows = (r0, r1)
        semas = (sema0, sema1)

        pltpu.sync_copy(col_hbm.at[wid], idx_all)

        def fill_zero(i, carry):
            for j in range(D // 16):
                r0[i, pl.ds(j * 16, 16)] = zeros16
            return carry

        lax.fori_loop(0, _SCH, fill_zero, 0)

        nbase = s * rpt
        for z in range(rpt // _SCH):
            pltpu.sync_copy(r0, sums_sh.at[pl.ds(nbase + z * _SCH, _SCH)])
        plsc.subcore_barrier()

        base = wid * per_w

        def do_chunk(kk, b):
            off = base + kk * _SCH
            pltpu.async_copy(h_hbm.at[pl.ds(off, _SCH)], rows[b], semh).wait()
            pltpu.async_copy(rows[b], sums_sh.at[idx_all.at[kk]],
                             semas[b], add=True)

        def add_wait(b):
            pltpu.make_async_copy(rows[b], sums_sh.at[idx_all.at[0]],
                                  semas[b]).wait()

        do_chunk(0, 0)
        do_chunk(1, 1)

        def body(j, carry):
            add_wait(0)
            do_chunk(2 * j, 0)
            add_wait(1)
            do_chunk(2 * j + 1, 1)
            return carry

        lax.fori_loop(1, nch // 2, body, 0)
        add_wait(0)
        add_wait(1)
        plsc.subcore_barrier()
        pltpu.sync_copy(sums_sh.at[pl.ds(nbase, rpt)],
                        sums_out.at[c, pl.ds(nbase, rpt)])

    return k(h, col2)


def _scatter_counts(col, E, D):
    per_w = E // _NW
    nch = per_w // _CCH
    rpt = _NPAD // _NS
    mesh = plsc.VectorSubcoreMesh(core_axis_name="c", subcore_axis_name="s")

    @functools.partial(
        pl.kernel,
        mesh=mesh,
        out_type=jax.ShapeDtypeStruct((_NC, _NPAD, D), jnp.float32),
        scratch_types=[
            pltpu.VMEM((nch, _CCH), jnp.int32),
            pltpu.VMEM((_CCH, D), jnp.float32),
            pltpu.VMEM_SHARED((_NPAD, D), jnp.float32),
            pltpu.SemaphoreType.DMA,
            pltpu.SemaphoreType.DMA,
        ],
    )
    def k(col_hbm, cnt_out, idx_all, ones_v, cnt_sh, sema0, sema1):
        c = lax.axis_index("c")
        s = lax.axis_index("s")
        wid = s * _NC + c
        zeros16 = jnp.zeros((16,), jnp.float32)
        ones16 = jnp.ones((16,), jnp.float32)
        semas = (sema0, sema1)

        pltpu.sync_copy(col_hbm.at[wid], idx_all)

        def fill_zero(i, carry):
            for j in range(D // 16):
                ones_v[i, pl.ds(j * 16, 16)] = zeros16
            return carry

        lax.fori_loop(0, _CCH, fill_zero, 0)

        nbase = s * rpt
        for z in range(rpt // _CCH):
            pltpu.sync_copy(ones_v, cnt_sh.at[pl.ds(nbase + z * _CCH, _CCH)])

        def fill_ones(i, carry):
            for j in range(D // 16):
                ones_v[i, pl.ds(j * 16, 16)] = ones16
            return carry

        lax.fori_loop(0, _CCH, fill_ones, 0)
        plsc.subcore_barrier()

        def do_chunk(kk, b):
            pltpu.async_copy(ones_v, cnt_sh.at[idx_all.at[kk]],
                             semas[b], add=True)

        def add_wait(b):
            pltpu.make_async_copy(ones_v, cnt_sh.at[idx_all.at[0]],
                                  semas[b]).wait()

        do_chunk(0, 0)
        do_chunk(1, 1)

        def body(j, carry):
            add_wait(0)
            do_chunk(2 * j, 0)
            add_wait(1)
            do_chunk(2 * j + 1, 1)
            return carry

        lax.fori_loop(1, nch // 2, body, 0)
        add_wait(0)
        do_chunk(nch - 1, 0)
        add_wait(1)
        add_wait(0)
        plsc.subcore_barrier()
        pltpu.sync_copy(cnt_sh.at[pl.ds(nbase, rpt)],
                        cnt_out.at[c, pl.ds(nbase, rpt)])

    return k(col)


def _mm_body(x_ref, w_ref, o_ref):
    o_ref[...] = jnp.dot(x_ref[...], w_ref[...],
                         preferred_element_type=jnp.float32)


def _edge_mlp_body(xg_ref, ea_ref, w1e_ref, b1a_ref, w1b_ref, b1b_ref, o_ref):
    h1 = xg_ref[...] + jnp.dot(ea_ref[...], w1e_ref[...],
                               preferred_element_type=jnp.float32)
    h1 = jnp.maximum(h1 + b1a_ref[...], 0.0)
    h2 = jnp.dot(h1, w1b_ref[...], preferred_element_type=jnp.float32)
    o_ref[...] = jnp.maximum(h2 + b1b_ref[...], 0.0)


def _node_mlp_body(x_ref, p0_ref, p1_ref, c0_ref, c1_ref, w2x_ref, w2a_ref,
                   b2a_ref, w2b_ref, b2b_ref, o_ref):
    cnt = c0_ref[0, :, 0:1] + c1_ref[0, :, 0:1]
    agg = (p0_ref[0] + p1_ref[0]) / jnp.maximum(cnt, 1.0)
    hid = jnp.dot(x_ref[...], w2x_ref[...], preferred_element_type=jnp.float32)
    hid = hid + jnp.dot(agg, w2a_ref[...], preferred_element_type=jnp.float32)
    hid = jnp.maximum(hid + b2a_ref[...], 0.0)
    out = jnp.dot(hid, w2b_ref[...], preferred_element_type=jnp.float32)
    o_ref[...] = out + b2b_ref[...]


def kernel(x, edge_index, edge_attr, W1a, b1a, W1b, b1b, W2a, b2a, W2b, b2b):
    N, D = x.shape
    E = edge_index.shape[1]
    DE = edge_attr.shape[1]
    row = edge_index[0].astype(jnp.int32)
    col = edge_index[1].astype(jnp.int32)

    W1a_x = W1a[:D]
    W1a_e = W1a[D:]
    W2a_x = W2a[:D]
    W2a_a = W2a[D:]
    b1a2 = b1a.reshape(1, -1)
    b1b2 = b1b.reshape(1, -1)
    b2a2 = b2a.reshape(1, -1)
    b2b2 = b2b.reshape(1, -1)

    col_c = col.reshape(_NW, E // _NW // _CCH, _CCH)
    col_s = col.reshape(_NW, E // _NW // _SCH, _SCH)
    row2 = row.reshape(_NW, E // _NW // _GCH, _GCH)

    x_pad = jnp.concatenate(
        [x, jnp.zeros((_NPAD - N, D), jnp.float32)], axis=0)
    PB = _NPAD // 8
    xw = pl.pallas_call(
        _mm_body,
        grid=(8,),
        in_specs=[
            pl.BlockSpec((PB, D), lambda i: (i, 0)),
            pl.BlockSpec((D, D), lambda i: (0, 0)),
        ],
        out_specs=pl.BlockSpec((PB, D), lambda i: (i, 0)),
        out_shape=jax.ShapeDtypeStruct((_NPAD, D), jnp.float32),
    )(x_pad, W1a_x)

    xg = _gather_rows(xw, row2, E, D)

    EB = 2560
    h = pl.pallas_call(
        _edge_mlp_body,
        grid=(E // EB,),
        in_specs=[
            pl.BlockSpec((EB, D), lambda i: (i, 0)),
            pl.BlockSpec((EB, DE), lambda i: (i, 0)),
            pl.BlockSpec((DE, D), lambda i: (0, 0)),
            pl.BlockSpec((1, D), lambda i: (0, 0)),
            pl.BlockSpec((D, D), lambda i: (0, 0)),
            pl.BlockSpec((1, D), lambda i: (0, 0)),
        ],
        out_specs=pl.BlockSpec((EB, D), lambda i: (i, 0)),
        out_shape=jax.ShapeDtypeStruct((E, D), jnp.float32),
    )(xg, edge_attr, W1a_e, b1a2, W1b, b1b2)

    cnt = _scatter_counts(col_c, E, D)

    sums = _scatter_sums(h, col_s, E, D)

    NB = 2000
    out = pl.pallas_call(
        _node_mlp_body,
        grid=(N // NB,),
        in_specs=[
            pl.BlockSpec((NB, D), lambda i: (i, 0)),
            pl.BlockSpec((1, NB, D), lambda i: (0, i, 0)),
            pl.BlockSpec((1, NB, D), lambda i: (1, i, 0)),
            pl.BlockSpec((1, NB, D), lambda i: (0, i, 0)),
            pl.BlockSpec((1, NB, D), lambda i: (1, i, 0)),
            pl.BlockSpec((D, D), lambda i: (0, 0)),
            pl.BlockSpec((D, D), lambda i: (0, 0)),
            pl.BlockSpec((1, D), lambda i: (0, 0)),
            pl.BlockSpec((D, D), lambda i: (0, 0)),
            pl.BlockSpec((1, D), lambda i: (0, 0)),
        ],
        out_specs=pl.BlockSpec((NB, D), lambda i: (i, 0)),
        out_shape=jax.ShapeDtypeStruct((N, D), jnp.float32),
    )(x, sums, sums, cnt, cnt, W2a_x, W2a_a, b2a2, W2b, b2b2)
    return out
